# baseline (speedup 1.0000x reference)
import math

import jax
import jax.numpy as jnp
from jax import lax
from jax.experimental import pallas as pl
from jax.experimental.pallas import tpu as pltpu


def _gelu_exact(x):
    return 0.5 * x * (1.0 + lax.erf(x * (1.0 / math.sqrt(2.0))))


_P = 8
_HT = 64
_TOP = 8
_HS = 80


def _conv_body(x_ref, a_ref, b_ref, o_ref, xp_ref):
    C, H, W = x_ref.shape[1], x_ref.shape[2], x_ref.shape[3]
    CH, WP = C * H, _P * W
    HP = xp_ref.shape[2]
    @pl.when(pl.program_id(0) == 0)
    def _zero_pad_rows():
        xp_ref[:, :, :_TOP] = jnp.zeros((3, C, _TOP, WP), xp_ref.dtype)
        xp_ref[:, :, _TOP + H :] = jnp.zeros(
            (3, C, HP - H - _TOP, WP), xp_ref.dtype
        )

    zcol = jnp.zeros((CH, 1), x_ref.dtype)
    for q in range(_P // 2):
        xi = x_ref[2 * q].reshape(CH, W)
        xj = x_ref[2 * q + 1].reshape(CH, W)
        pair = jnp.concatenate([xi, xj], axis=1)
        pair_m = jnp.concatenate(
            [zcol, xi[:, : W - 1], zcol, xj[:, : W - 1]], axis=1
        )
        pair_p = jnp.concatenate(
            [xi[:, 1:], zcol, xj[:, 1:], zcol], axis=1
        )
        for kx, v in enumerate((pair_m, pair, pair_p)):
            xp_ref[kx, :, _TOP : _TOP + H, 2 * q * W : 2 * (q + 1) * W] = (
                v.astype(xp_ref.dtype).reshape(C, H, 2 * W)
            )

    for h0 in list(range(0, H - _HT, _HT)) + [H - _HT]:
        slab = xp_ref[:, :, h0 : h0 + _HS, :].reshape(3 * C * _HS, WP)
        acc = jnp.dot(a_ref[...], slab, preferred_element_type=jnp.float32)
        y = _gelu_exact(acc.reshape(C, _HT, WP) + b_ref[...][:, None, :])
        y = y.astype(o_ref.dtype)
        for p in range(_P):
            o_ref[p, :, h0 : h0 + _HT, :] = y[:, :, p * W : (p + 1) * W]


def _build_tap_mats(weight_oihw, ht, hs):
    C = weight_oihw.shape[0]
    w = weight_oihw.astype(jnp.float32)
    mats = []
    for kx in range(3):
        m = jnp.zeros((C * ht, C * hs), jnp.float32)
        for ky in range(3):
            eye = jnp.eye(ht, hs, k=ky + _TOP - 1, dtype=jnp.float32)
            m = m + jnp.einsum("oi,dp->odip", w[:, :, ky, kx], eye).reshape(
                C * ht, C * hs
            )
        mats.append(m)
    return jnp.concatenate(mats, axis=1).astype(jnp.bfloat16)


def _invariant(block_shape):
    index_map = lambda i: (0,) * len(block_shape)
    if hasattr(pl, "Buffered"):
        try:
            return pl.BlockSpec(block_shape, index_map, pipeline_mode=pl.Buffered(1))
        except TypeError:
            pass
    return pl.BlockSpec(block_shape, index_map)


def kernel(x_nchw, weight_oihw, bias):
    N, C, H, W = x_nchw.shape
    a_mats = _build_tap_mats(weight_oihw, _HT, _HS)
    bias_mat = jnp.broadcast_to(bias.astype(jnp.float32)[:, None], (C, _P * W))
    hp = max(H - _HT + _HS, H + 2 * _TOP)

    return pl.pallas_call(
        _conv_body,
        out_shape=jax.ShapeDtypeStruct((N, C, H, W), x_nchw.dtype),
        grid=(N // _P,),
        in_specs=[
            pl.BlockSpec((_P, C, H, W), lambda i: (i, 0, 0, 0)),
            _invariant((C * _HT, 3 * C * _HS)),
            _invariant((C, _P * W)),
        ],
        out_specs=pl.BlockSpec((_P, C, H, W), lambda i: (i, 0, 0, 0)),
        scratch_shapes=[pltpu.VMEM((3, C, hp, _P * W), jnp.bfloat16)],
        compiler_params=pltpu.CompilerParams(
            dimension_semantics=("parallel",),
            vmem_limit_bytes=60 * 1024 * 1024,
        ),
    )(x_nchw, a_mats, bias_mat)


# speedup vs baseline: 1.5621x; 1.5621x over previous
import math

import jax
import jax.numpy as jnp
from jax import lax
from jax.experimental import pallas as pl
from jax.experimental.pallas import tpu as pltpu


def _gelu_exact(x):
    return 0.5 * x * (1.0 + lax.erf(x * (1.0 / math.sqrt(2.0))))


_P = 16
_HT = 32
_TOP = 8
_HS = 48


def _conv_body(x_ref, a_ref, b_ref, o_ref, xp_ref):
    C, H, W = x_ref.shape[1], x_ref.shape[2], x_ref.shape[3]
    CH, WP = C * H, _P * W
    HP = xp_ref.shape[2]
    @pl.when(pl.program_id(0) == 0)
    def _zero_pad_rows():
        xp_ref[:, :, :_TOP] = jnp.zeros((3, C, _TOP, WP), xp_ref.dtype)
        xp_ref[:, :, _TOP + H :] = jnp.zeros(
            (3, C, HP - H - _TOP, WP), xp_ref.dtype
        )

    zcol = jnp.zeros((CH, 1), x_ref.dtype)
    for q in range(_P // 2):
        xi = x_ref[2 * q].reshape(CH, W)
        xj = x_ref[2 * q + 1].reshape(CH, W)
        pair = jnp.concatenate([xi, xj], axis=1)
        pair_m = jnp.concatenate(
            [zcol, xi[:, : W - 1], zcol, xj[:, : W - 1]], axis=1
        )
        pair_p = jnp.concatenate(
            [xi[:, 1:], zcol, xj[:, 1:], zcol], axis=1
        )
        for kx, v in enumerate((pair_m, pair, pair_p)):
            xp_ref[kx, :, _TOP : _TOP + H, 2 * q * W : 2 * (q + 1) * W] = (
                v.astype(xp_ref.dtype).reshape(C, H, 2 * W)
            )

    for h0 in list(range(0, H - _HT, _HT)) + [H - _HT]:
        slab = xp_ref[:, :, h0 : h0 + _HS, :].reshape(3 * C * _HS, WP)
        acc = jnp.dot(a_ref[...], slab, preferred_element_type=jnp.float32)
        y = _gelu_exact(acc.reshape(C, _HT, WP) + b_ref[...][:, None, :])
        y = y.astype(o_ref.dtype)
        for p in range(_P):
            o_ref[p, :, h0 : h0 + _HT, :] = y[:, :, p * W : (p + 1) * W]


def _build_tap_mats(weight_oihw, ht, hs):
    C = weight_oihw.shape[0]
    w = weight_oihw.astype(jnp.float32)
    mats = []
    for kx in range(3):
        m = jnp.zeros((C * ht, C * hs), jnp.float32)
        for ky in range(3):
            eye = jnp.eye(ht, hs, k=ky + _TOP - 1, dtype=jnp.float32)
            m = m + jnp.einsum("oi,dp->odip", w[:, :, ky, kx], eye).reshape(
                C * ht, C * hs
            )
        mats.append(m)
    return jnp.concatenate(mats, axis=1).astype(jnp.bfloat16)


def _invariant(block_shape):
    index_map = lambda i: (0,) * len(block_shape)
    if hasattr(pl, "Buffered"):
        try:
            return pl.BlockSpec(block_shape, index_map, pipeline_mode=pl.Buffered(1))
        except TypeError:
            pass
    return pl.BlockSpec(block_shape, index_map)


def kernel(x_nchw, weight_oihw, bias):
    N, C, H, W = x_nchw.shape
    a_mats = _build_tap_mats(weight_oihw, _HT, _HS)
    bias_mat = jnp.broadcast_to(bias.astype(jnp.float32)[:, None], (C, _P * W))
    hp = max(H - _HT + _HS, H + 2 * _TOP)

    return pl.pallas_call(
        _conv_body,
        out_shape=jax.ShapeDtypeStruct((N, C, H, W), x_nchw.dtype),
        grid=(N // _P,),
        in_specs=[
            pl.BlockSpec((_P, C, H, W), lambda i: (i, 0, 0, 0)),
            _invariant((C * _HT, 3 * C * _HS)),
            _invariant((C, _P * W)),
        ],
        out_specs=pl.BlockSpec((_P, C, H, W), lambda i: (i, 0, 0, 0)),
        scratch_shapes=[pltpu.VMEM((3, C, hp, _P * W), jnp.bfloat16)],
        compiler_params=pltpu.CompilerParams(
            dimension_semantics=("parallel",),
            vmem_limit_bytes=60 * 1024 * 1024,
        ),
    )(x_nchw, a_mats, bias_mat)
